# baseline (speedup 1.0000x reference)
import dataclasses
import functools
import math

import jax
import jax.numpy as jnp
from jax.experimental import pallas as pl
from jax.experimental.pallas import tpu as pltpu
from jax.experimental.pallas import tpu_sc as plsc

N = 10000
D = 128
E = 320000
STEPS = 4

NC = 2
NS = 16
NW = NC * NS
EPW = E // NW
CHUNK = 40
NCH = EPW // CHUNK
RPT = 624
LANES = 16
SCALE = 1.0 / math.sqrt(float(D))


def _mm5_body(x_ref, w_ref, o0, o1, o2, o3, o4):
    y = jnp.dot(x_ref[...], w_ref[...], preferred_element_type=jnp.float32)
    o0[...] = y[:, 0 * D:1 * D]
    o1[...] = y[:, 1 * D:2 * D]
    o2[...] = y[:, 2 * D:3 * D]
    o3[...] = y[:, 3 * D:4 * D]
    o4[...] = y[:, 4 * D:5 * D]


_MM_BLK = 1000


def _mm5(x, wcat):
    out = jax.ShapeDtypeStruct((N, D), jnp.float32)
    return pl.pallas_call(
        _mm5_body,
        grid=(N // _MM_BLK,),
        in_specs=[
            pl.BlockSpec((_MM_BLK, D), lambda i: (i, 0)),
            pl.BlockSpec((D, 5 * D), lambda i: (0, 0)),
        ],
        out_specs=[pl.BlockSpec((_MM_BLK, D), lambda i: (i, 0))] * 5,
        out_shape=[out] * 5,
    )(x, wcat)


def _mm5c_body(xw0_ref, p_ref, w_ref, o0, o1, o2, o3, o4):
    xb = xw0_ref[...] + p_ref[0] + p_ref[1]
    y = jnp.dot(xb, w_ref[...], preferred_element_type=jnp.float32)
    o0[...] = y[:, 0 * D:1 * D]
    o1[...] = y[:, 1 * D:2 * D]
    o2[...] = y[:, 2 * D:3 * D]
    o3[...] = y[:, 3 * D:4 * D]
    o4[...] = y[:, 4 * D:5 * D]


def _mm5c(xw0, parts, wcat):
    out = jax.ShapeDtypeStruct((N, D), jnp.float32)
    return pl.pallas_call(
        _mm5c_body,
        grid=(N // _MM_BLK,),
        in_specs=[
            pl.BlockSpec((_MM_BLK, D), lambda i: (i, 0)),
            pl.BlockSpec((NC, _MM_BLK, D), lambda i: (0, i, 0)),
            pl.BlockSpec((D, 5 * D), lambda i: (0, 0)),
        ],
        out_specs=[pl.BlockSpec((_MM_BLK, D), lambda i: (i, 0))] * 5,
        out_shape=[out] * 5,
    )(xw0, parts, wcat)


def _combine_body(xw0_ref, p_ref, o_ref):
    o_ref[...] = xw0_ref[...] + p_ref[0] + p_ref[1]


def _combine(xw0, parts):
    return pl.pallas_call(
        _combine_body,
        grid=(N // _MM_BLK,),
        in_specs=[
            pl.BlockSpec((_MM_BLK, D), lambda i: (i, 0)),
            pl.BlockSpec((NC, _MM_BLK, D), lambda i: (0, i, 0)),
        ],
        out_specs=pl.BlockSpec((_MM_BLK, D), lambda i: (i, 0)),
        out_shape=jax.ShapeDtypeStruct((N, D), jnp.float32),
    )(xw0, parts)


NSET = 4
NBUF = 2


def _sc_edge_body(y1_hbm, q_hbm, k_hbm, v_hbm, src_hbm, dst_hbm, d2_hbm,
                  s2_hbm, out_hbm, *sc):
    idx1s = sc[0:NSET]
    idx1d = sc[NSET:2 * NSET]
    idx2d = sc[2 * NSET:3 * NSET]
    idx2s = sc[3 * NSET:4 * NSET]
    rows1 = sc[16:18]
    rowsq = sc[18:20]
    rowsk = sc[20:22]
    rowsv = sc[22:24]
    acc = sc[24]
    _s0 = 25
    semA = sc[_s0:_s0 + NSET]
    g1 = sc[_s0 + NSET:_s0 + NSET + 2]
    g2 = sc[_s0 + NSET + 2:_s0 + NSET + 4]
    sc1 = sc[_s0 + NSET + 4:_s0 + NSET + 6]
    sc2 = sc[_s0 + NSET + 6:_s0 + NSET + 8]

    c = jax.lax.axis_index("c")
    s = jax.lax.axis_index("s")
    wid = s * NC + c
    ebase = wid * EPW

    zeros16 = jnp.zeros((LANES,), jnp.float32)

    @pl.loop(0, CHUNK)
    def _(e):
        for j in range(D // LANES):
            rows1[0][e, pl.ds(j * LANES, LANES)] = zeros16

    row0 = s * RPT
    zcopies = []
    for ci in range(RPT // CHUNK):
        zcopies.append(pltpu.async_copy(
            rows1[0], acc.at[pl.ds(row0 + ci * CHUNK, CHUNK)], g1[0]))
    rem = RPT % CHUNK
    if rem:
        zcopies.append(pltpu.async_copy(
            rows1[0].at[pl.ds(0, rem)],
            acc.at[pl.ds(row0 + (RPT // CHUNK) * CHUNK, rem)], g1[0]))
    for cp_ in zcopies:
        cp_.wait()

    @pl.when(s == NS - 1)
    def _():
        pltpu.sync_copy(rows1[0].at[pl.ds(0, N - NS * RPT)],
                        acc.at[pl.ds(NS * RPT, N - NS * RPT)])

    plsc.subcore_barrier()

    def issue_idx(m, j):
        off = ebase + m * CHUNK
        pltpu.async_copy(src_hbm.at[pl.ds(off, CHUNK)], idx1s[j], semA[j])
        pltpu.async_copy(dst_hbm.at[pl.ds(off, CHUNK)], idx1d[j], semA[j])
        pltpu.async_copy(d2_hbm.at[pl.ds(off, CHUNK)], idx2d[j], semA[j])
        pltpu.async_copy(s2_hbm.at[pl.ds(off, CHUNK)], idx2s[j], semA[j])

    def wait_idx(j):
        pltpu.make_async_copy(src_hbm.at[pl.ds(0, CHUNK)], idx1s[j], semA[j]).wait()
        pltpu.make_async_copy(dst_hbm.at[pl.ds(0, CHUNK)], idx1d[j], semA[j]).wait()
        pltpu.make_async_copy(d2_hbm.at[pl.ds(0, CHUNK)], idx2d[j], semA[j]).wait()
        pltpu.make_async_copy(s2_hbm.at[pl.ds(0, CHUNK)], idx2s[j], semA[j]).wait()

    def issue_gathers(j, b):
        pltpu.async_copy(q_hbm.at[idx2d[j]], rowsq[b], g2[b])
        pltpu.async_copy(k_hbm.at[idx2s[j]], rowsk[b], g2[b])
        pltpu.async_copy(v_hbm.at[idx2s[j]], rowsv[b], g2[b])
        pltpu.async_copy(y1_hbm.at[idx1s[j]], rows1[b], g1[b])

    def wait_gather_y1(j, b):
        pltpu.make_async_copy(y1_hbm.at[idx1s[j]], rows1[b], g1[b]).wait()

    def wait_gathers_qkv(j, b):
        pltpu.make_async_copy(q_hbm.at[idx2d[j]], rowsq[b], g2[b]).wait()
        pltpu.make_async_copy(k_hbm.at[idx2s[j]], rowsk[b], g2[b]).wait()
        pltpu.make_async_copy(v_hbm.at[idx2s[j]], rowsv[b], g2[b]).wait()

    def issue_scatter1(j, b):
        pltpu.async_copy(rows1[b], acc.at[idx1d[j]], sc1[b], add=True)

    def issue_scatter2(j, b):
        pltpu.async_copy(rowsv[b], acc.at[idx2d[j]], sc2[b], add=True)

    def wait_scatters(j, b):
        pltpu.make_async_copy(rows1[b], acc.at[idx1d[j]], sc1[b]).wait()
        pltpu.make_async_copy(rowsv[b], acc.at[idx2d[j]], sc2[b]).wait()

    def compute(b):
        @pl.loop(0, CHUNK, step=2)
        def _(e0):
            for u in range(2):
                e = e0 + u
                dotv = rowsq[b][e, pl.ds(0, LANES)] * rowsk[b][e, pl.ds(0, LANES)]
                for j in range(1, D // LANES):
                    dotv += (rowsq[b][e, pl.ds(j * LANES, LANES)]
                             * rowsk[b][e, pl.ds(j * LANES, LANES)])
                alpha = jnp.sum(dotv) * (-SCALE)
                for j in range(D // LANES):
                    sl = pl.ds(j * LANES, LANES)
                    rowsv[b][e, sl] = rowsv[b][e, sl] * alpha

    issue_idx(0, 0)
    issue_idx(1, 1)
    wait_idx(0)
    issue_gathers(0, 0)

    def guarded(cond, fn):
        if isinstance(cond, bool):
            if cond:
                fn()
        else:
            pl.when(cond)(fn)

    def body(i, k):
        b = k % NBUF
        bn = (k + 1) % NBUF
        j = k % NSET
        jn = (k + 1) % NSET
        jn2 = (k + 2) % NSET
        jp = (k - 1) % NSET

        guarded(i < NCH - 2, lambda: issue_idx(i + 2, jn2))
        guarded(i > 0, lambda: wait_scatters(jp, bn))

        def _prefetch_rows():
            wait_idx(jn)
            issue_gathers(jn, bn)

        guarded(i < NCH - 1, _prefetch_rows)

        wait_gather_y1(j, b)
        issue_scatter1(j, b)
        wait_gathers_qkv(j, b)
        compute(b)
        issue_scatter2(j, b)

    _M = ((NCH - 1) // NSET) * NSET

    @pl.loop(0, _M, step=NSET)
    def _(g):
        for k in range(NSET):
            body(g + k, k)

    for i in range(_M, NCH):
        body(i, i % NSET)
    wait_scatters((NCH - 1) % NSET, (NCH - 1) % NBUF)

    plsc.subcore_barrier()

    pltpu.sync_copy(acc.at[pl.ds(row0, RPT)], out_hbm.at[c].at[pl.ds(row0, RPT)])

    @pl.when(s == NS - 1)
    def _():
        pltpu.sync_copy(acc.at[pl.ds(NS * RPT, N - NS * RPT)],
                        out_hbm.at[c].at[pl.ds(NS * RPT, N - NS * RPT)])


@jax.jit
def _sc_edge(y1, q, k, v, src, dst, d2, s2):
    mesh = plsc.VectorSubcoreMesh(core_axis_name="c", subcore_axis_name="s")
    cp = pltpu.CompilerParams()
    if "needs_layout_passes" in pltpu.CompilerParams.__dataclass_fields__:
        cp = dataclasses.replace(cp, needs_layout_passes=False)
    f = pl.kernel(
        _sc_edge_body,
        out_type=jax.ShapeDtypeStruct((NC, N, D), jnp.float32),
        mesh=mesh,
        scratch_types=(
            [pltpu.VMEM((CHUNK,), jnp.int32)] * (4 * NSET)
            + [pltpu.VMEM((CHUNK, D), jnp.float32)] * (4 * NBUF)
            + [pltpu.VMEM_SHARED((N, D), jnp.float32)]
            + [pltpu.SemaphoreType.DMA] * (NSET + 4 * NBUF)
        ),
        compiler_params=cp,
    )
    return f(y1, q, k, v, src, dst, d2, s2)


def kernel(input, edge_index, edge_index_2, W0, W1, Wq, Wk, Wv):
    x = input
    src = edge_index[0].astype(jnp.int32)
    dst = edge_index[1].astype(jnp.int32)
    s2 = edge_index_2[0].astype(jnp.int32)
    d2 = edge_index_2[1].astype(jnp.int32)
    wcat = jnp.concatenate([W0, W1, Wq, Wk, Wv], axis=1)
    xw0, y1, q, k, v = _mm5(x, wcat)
    parts = _sc_edge(y1, q, k, v, src, dst, d2, s2)
    for _ in range(STEPS - 1):
        xw0, y1, q, k, v = _mm5c(xw0, parts, wcat)
        parts = _sc_edge(y1, q, k, v, src, dst, d2, s2)
    return _combine(xw0, parts)


# speedup vs baseline: 1.0022x; 1.0022x over previous
import dataclasses
import functools
import math

import jax
import jax.numpy as jnp
from jax.experimental import pallas as pl
from jax.experimental.pallas import tpu as pltpu
from jax.experimental.pallas import tpu_sc as plsc

N = 10000
D = 128
E = 320000
STEPS = 4

NC = 2
NS = 16
NW = NC * NS
EPW = E // NW
CHUNK = 40
NCH = EPW // CHUNK
RPT = 624
LANES = 16
SCALE = 1.0 / math.sqrt(float(D))


def _mm5_body(x_ref, w_ref, o0, o1, o2, o3, o4):
    y = jnp.dot(x_ref[...], w_ref[...], preferred_element_type=jnp.float32)
    o0[...] = y[:, 0 * D:1 * D]
    o1[...] = y[:, 1 * D:2 * D]
    o2[...] = y[:, 2 * D:3 * D]
    o3[...] = y[:, 3 * D:4 * D]
    o4[...] = y[:, 4 * D:5 * D]


_MM_BLK = 1000


def _mm5(x, wcat):
    out = jax.ShapeDtypeStruct((N, D), jnp.float32)
    return pl.pallas_call(
        _mm5_body,
        grid=(N // _MM_BLK,),
        in_specs=[
            pl.BlockSpec((_MM_BLK, D), lambda i: (i, 0)),
            pl.BlockSpec((D, 5 * D), lambda i: (0, 0)),
        ],
        out_specs=[pl.BlockSpec((_MM_BLK, D), lambda i: (i, 0))] * 5,
        out_shape=[out] * 5,
    )(x, wcat)


def _mm5c_body(xw0_ref, p_ref, w_ref, o0, o1, o2, o3, o4):
    xb = xw0_ref[...] + p_ref[0] + p_ref[1]
    y = jnp.dot(xb, w_ref[...], preferred_element_type=jnp.float32)
    o0[...] = y[:, 0 * D:1 * D]
    o1[...] = y[:, 1 * D:2 * D]
    o2[...] = y[:, 2 * D:3 * D]
    o3[...] = y[:, 3 * D:4 * D]
    o4[...] = y[:, 4 * D:5 * D]


def _mm5c(xw0, parts, wcat):
    out = jax.ShapeDtypeStruct((N, D), jnp.float32)
    return pl.pallas_call(
        _mm5c_body,
        grid=(N // _MM_BLK,),
        in_specs=[
            pl.BlockSpec((_MM_BLK, D), lambda i: (i, 0)),
            pl.BlockSpec((NC, _MM_BLK, D), lambda i: (0, i, 0)),
            pl.BlockSpec((D, 5 * D), lambda i: (0, 0)),
        ],
        out_specs=[pl.BlockSpec((_MM_BLK, D), lambda i: (i, 0))] * 5,
        out_shape=[out] * 5,
    )(xw0, parts, wcat)


def _combine_body(xw0_ref, p_ref, o_ref):
    o_ref[...] = xw0_ref[...] + p_ref[0] + p_ref[1]


def _combine(xw0, parts):
    return pl.pallas_call(
        _combine_body,
        grid=(N // _MM_BLK,),
        in_specs=[
            pl.BlockSpec((_MM_BLK, D), lambda i: (i, 0)),
            pl.BlockSpec((NC, _MM_BLK, D), lambda i: (0, i, 0)),
        ],
        out_specs=pl.BlockSpec((_MM_BLK, D), lambda i: (i, 0)),
        out_shape=jax.ShapeDtypeStruct((N, D), jnp.float32),
    )(xw0, parts)


NSET = 4
NBUF = 2


def _sc_edge_body(y1_hbm, q_hbm, k_hbm, v_hbm, src_hbm, dst_hbm, d2_hbm,
                  s2_hbm, out_hbm, *sc):
    idx1s = sc[0:NSET]
    idx1d = sc[NSET:2 * NSET]
    idx2d = sc[2 * NSET:3 * NSET]
    idx2s = sc[3 * NSET:4 * NSET]
    rows1 = sc[16:18]
    rowsq = sc[18:20]
    rowsk = sc[20:22]
    rowsv = sc[22:24]
    acc = sc[24]
    _s0 = 25
    semA = sc[_s0:_s0 + NSET]
    g1 = sc[_s0 + NSET:_s0 + NSET + 2]
    g2 = sc[_s0 + NSET + 2:_s0 + NSET + 4]
    sc1 = sc[_s0 + NSET + 4:_s0 + NSET + 6]
    sc2 = sc[_s0 + NSET + 6:_s0 + NSET + 8]

    c = jax.lax.axis_index("c")
    s = jax.lax.axis_index("s")
    wid = s * NC + c
    ebase = wid * EPW

    zeros16 = jnp.zeros((LANES,), jnp.float32)

    @pl.loop(0, CHUNK)
    def _(e):
        for j in range(D // LANES):
            rows1[0][e, pl.ds(j * LANES, LANES)] = zeros16

    row0 = s * RPT
    zcopies = []
    for ci in range(RPT // CHUNK):
        zcopies.append(pltpu.async_copy(
            rows1[0], acc.at[pl.ds(row0 + ci * CHUNK, CHUNK)], g1[0]))
    rem = RPT % CHUNK
    if rem:
        zcopies.append(pltpu.async_copy(
            rows1[0].at[pl.ds(0, rem)],
            acc.at[pl.ds(row0 + (RPT // CHUNK) * CHUNK, rem)], g1[0]))
    for cp_ in zcopies:
        cp_.wait()

    @pl.when(s == NS - 1)
    def _():
        pltpu.sync_copy(rows1[0].at[pl.ds(0, N - NS * RPT)],
                        acc.at[pl.ds(NS * RPT, N - NS * RPT)])

    plsc.subcore_barrier()

    def issue_idx(m, j):
        off = ebase + m * CHUNK
        pltpu.async_copy(src_hbm.at[pl.ds(off, CHUNK)], idx1s[j], semA[j])
        pltpu.async_copy(dst_hbm.at[pl.ds(off, CHUNK)], idx1d[j], semA[j])
        pltpu.async_copy(d2_hbm.at[pl.ds(off, CHUNK)], idx2d[j], semA[j])
        pltpu.async_copy(s2_hbm.at[pl.ds(off, CHUNK)], idx2s[j], semA[j])

    def wait_idx(j):
        pltpu.make_async_copy(src_hbm.at[pl.ds(0, CHUNK)], idx1s[j], semA[j]).wait()
        pltpu.make_async_copy(dst_hbm.at[pl.ds(0, CHUNK)], idx1d[j], semA[j]).wait()
        pltpu.make_async_copy(d2_hbm.at[pl.ds(0, CHUNK)], idx2d[j], semA[j]).wait()
        pltpu.make_async_copy(s2_hbm.at[pl.ds(0, CHUNK)], idx2s[j], semA[j]).wait()

    def issue_gathers(j, b):
        pltpu.async_copy(q_hbm.at[idx2d[j]], rowsq[b], g2[b])
        pltpu.async_copy(k_hbm.at[idx2s[j]], rowsk[b], g2[b])
        pltpu.async_copy(v_hbm.at[idx2s[j]], rowsv[b], g2[b])
        pltpu.async_copy(y1_hbm.at[idx1s[j]], rows1[b], g1[b])

    def wait_gather_y1(j, b):
        pltpu.make_async_copy(y1_hbm.at[idx1s[j]], rows1[b], g1[b]).wait()

    def wait_gathers_qkv(j, b):
        pltpu.make_async_copy(q_hbm.at[idx2d[j]], rowsq[b], g2[b]).wait()
        pltpu.make_async_copy(k_hbm.at[idx2s[j]], rowsk[b], g2[b]).wait()
        pltpu.make_async_copy(v_hbm.at[idx2s[j]], rowsv[b], g2[b]).wait()

    def issue_scatter1(j, b):
        pltpu.async_copy(rows1[b], acc.at[idx1d[j]], sc1[b], add=True)

    def issue_scatter2(j, b):
        pltpu.async_copy(rowsv[b], acc.at[idx2d[j]], sc2[b], add=True)

    def wait_scatters(j, b):
        pltpu.make_async_copy(rows1[b], acc.at[idx1d[j]], sc1[b]).wait()
        pltpu.make_async_copy(rowsv[b], acc.at[idx2d[j]], sc2[b]).wait()

    def compute(b):
        @pl.loop(0, CHUNK, step=2)
        def _(e0):
            for u in range(2):
                e = e0 + u
                dotv = rowsq[b][e, pl.ds(0, LANES)] * rowsk[b][e, pl.ds(0, LANES)]
                for j in range(1, D // LANES):
                    dotv += (rowsq[b][e, pl.ds(j * LANES, LANES)]
                             * rowsk[b][e, pl.ds(j * LANES, LANES)])
                alpha = jnp.sum(dotv) * (-SCALE)
                for j in range(D // LANES):
                    sl = pl.ds(j * LANES, LANES)
                    rowsv[b][e, sl] = rowsv[b][e, sl] * alpha

    issue_idx(0, 0)
    issue_idx(1, 1)
    wait_idx(0)
    issue_gathers(0, 0)

    def guarded(cond, fn):
        if isinstance(cond, bool):
            if cond:
                fn()
        else:
            pl.when(cond)(fn)

    def body(i, k):
        b = k % NBUF
        bn = (k + 1) % NBUF
        j = k % NSET
        jn = (k + 1) % NSET
        jn2 = (k + 2) % NSET
        jp = (k - 1) % NSET

        guarded(i > 0, lambda: wait_scatters(jp, bn))

        def _prefetch_rows():
            wait_idx(jn)
            issue_gathers(jn, bn)

        guarded(i < NCH - 1, _prefetch_rows)
        guarded(i < NCH - 2, lambda: issue_idx(i + 2, jn2))

        wait_gather_y1(j, b)
        issue_scatter1(j, b)
        wait_gathers_qkv(j, b)
        compute(b)
        issue_scatter2(j, b)

    _M = ((NCH - 1) // NSET) * NSET

    @pl.loop(0, _M, step=NSET)
    def _(g):
        for k in range(NSET):
            body(g + k, k)

    for i in range(_M, NCH):
        body(i, i % NSET)
    wait_scatters((NCH - 1) % NSET, (NCH - 1) % NBUF)

    plsc.subcore_barrier()

    pltpu.sync_copy(acc.at[pl.ds(row0, RPT)], out_hbm.at[c].at[pl.ds(row0, RPT)])

    @pl.when(s == NS - 1)
    def _():
        pltpu.sync_copy(acc.at[pl.ds(NS * RPT, N - NS * RPT)],
                        out_hbm.at[c].at[pl.ds(NS * RPT, N - NS * RPT)])


@jax.jit
def _sc_edge(y1, q, k, v, src, dst, d2, s2):
    mesh = plsc.VectorSubcoreMesh(core_axis_name="c", subcore_axis_name="s")
    cp = pltpu.CompilerParams()
    if "needs_layout_passes" in pltpu.CompilerParams.__dataclass_fields__:
        cp = dataclasses.replace(cp, needs_layout_passes=False)
    f = pl.kernel(
        _sc_edge_body,
        out_type=jax.ShapeDtypeStruct((NC, N, D), jnp.float32),
        mesh=mesh,
        scratch_types=(
            [pltpu.VMEM((CHUNK,), jnp.int32)] * (4 * NSET)
            + [pltpu.VMEM((CHUNK, D), jnp.float32)] * (4 * NBUF)
            + [pltpu.VMEM_SHARED((N, D), jnp.float32)]
            + [pltpu.SemaphoreType.DMA] * (NSET + 4 * NBUF)
        ),
        compiler_params=cp,
    )
    return f(y1, q, k, v, src, dst, d2, s2)


def kernel(input, edge_index, edge_index_2, W0, W1, Wq, Wk, Wv):
    x = input
    src = edge_index[0].astype(jnp.int32)
    dst = edge_index[1].astype(jnp.int32)
    s2 = edge_index_2[0].astype(jnp.int32)
    d2 = edge_index_2[1].astype(jnp.int32)
    wcat = jnp.concatenate([W0, W1, Wq, Wk, Wv], axis=1)
    xw0, y1, q, k, v = _mm5(x, wcat)
    parts = _sc_edge(y1, q, k, v, src, dst, d2, s2)
    for _ in range(STEPS - 1):
        xw0, y1, q, k, v = _mm5c(xw0, parts, wcat)
        parts = _sc_edge(y1, q, k, v, src, dst, d2, s2)
    return _combine(xw0, parts)
